# Initial kernel scaffold; baseline (speedup 1.0000x reference)
import jax
import jax.numpy as jnp
from jax.experimental import pallas as pl


def kernel(input, target):
    raise NotImplementedError("write your pallas kernel here")


# speedup vs baseline: 183.1886x; 183.1886x over previous
import jax
import jax.numpy as jnp
from jax.experimental import pallas as pl
from jax.experimental.pallas import tpu as pltpu

_C = 19
_B, _H, _W = 4, 512, 512
_HB = 128
_NPIX = _B * _H * _W


def _loss_kernel(x_ref, t_ref, out_ref, acc_ref):
    step = pl.program_id(0)

    @pl.when(step == 0)
    def _init():
        acc_ref[...] = jnp.zeros_like(acc_ref)

    t = t_ref[0]
    m = x_ref[0, 0]
    best = jnp.zeros_like(t)
    lt = jnp.where(t == 0, m, 0.0)
    for c in range(1, _C):
        v = x_ref[0, c]
        gt_mask = v > m
        best = jnp.where(gt_mask, c, best)
        m = jnp.where(gt_mask, v, m)
        lt = jnp.where(t == c, v, lt)
    s = jnp.exp(x_ref[0, 0] - m)
    for c in range(1, _C):
        s = s + jnp.exp(x_ref[0, c] - m)
    ce = jnp.log(s) + m - lt
    mism = (best != t).astype(jnp.float32)
    for c in range(_C):
        maskf = (t == c).astype(jnp.float32)
        acc_ref[pl.ds(c, 1), :] += jnp.sum(maskf, axis=0, keepdims=True)
        acc_ref[pl.ds(_C + c, 1), :] += jnp.sum(maskf * mism, axis=0,
                                                keepdims=True)
        acc_ref[pl.ds(2 * _C + c, 1), :] += jnp.sum(maskf * ce, axis=0,
                                                    keepdims=True)

    @pl.when(step == pl.num_programs(0) - 1)
    def _fin():
        gt = jnp.sum(acc_ref[pl.ds(0, _C), :], axis=1)
        fn = jnp.sum(acc_ref[pl.ds(_C, _C), :], axis=1)
        ces = jnp.sum(acc_ref[pl.ds(2 * _C, _C), :], axis=1)
        w = jnp.where(fn > 0, fn, 1.0) / jnp.where(gt > 0, gt, 1.0)
        out_ref[...] = jnp.reshape(jnp.sum(w * ces) / _NPIX, (1, 1))


def kernel(input, target):
    nh = _H // _HB
    grid = (_B * nh,)
    out = pl.pallas_call(
        _loss_kernel,
        grid=grid,
        in_specs=[
            pl.BlockSpec((1, _C, _HB, _W), lambda i: (i // nh, 0, i % nh, 0)),
            pl.BlockSpec((1, _HB, _W), lambda i: (i // nh, i % nh, 0)),
        ],
        out_specs=pl.BlockSpec((1, 1), lambda i: (0, 0)),
        out_shape=jax.ShapeDtypeStruct((1, 1), jnp.float32),
        scratch_shapes=[pltpu.VMEM((64, _W), jnp.float32)],
        compiler_params=pltpu.CompilerParams(
            dimension_semantics=("arbitrary",),
        ),
    )(input, target)
    return out[0, 0]


# speedup vs baseline: 256.4573x; 1.4000x over previous
import jax
import jax.numpy as jnp
from jax.experimental import pallas as pl
from jax.experimental.pallas import tpu as pltpu

_C = 19
_B, _H, _W = 4, 512, 512
_HB = 128
_NPIX = _B * _H * _W


def _loss_kernel(x_ref, t_ref, out_ref, acc_ref):
    step = pl.program_id(0)

    @pl.when(step == 0)
    def _init():
        acc_ref[...] = jnp.zeros_like(acc_ref)

    t = t_ref[0]
    m = x_ref[0, 0]
    for c in range(1, _C):
        m = jnp.maximum(m, x_ref[0, c])
    first = jnp.full(t.shape, _C, jnp.int32)
    s = jnp.zeros_like(m)
    for c in range(_C - 1, -1, -1):
        v = x_ref[0, c]
        s = s + jnp.exp(v - m)
        first = jnp.where(v == m, c, first)
    lse = jnp.log(s) + m
    mism = (first != t).astype(jnp.float32)
    for c in range(_C):
        maskf = (t == c).astype(jnp.float32)
        acc_ref[c] += _rs(maskf)
        acc_ref[_C + c] += _rs(maskf * mism)
        acc_ref[2 * _C + c] += _rs(maskf * (lse - x_ref[0, c]))

    @pl.when(step == pl.num_programs(0) - 1)
    def _fin():
        gt = jnp.sum(acc_ref[pl.ds(0, _C)], axis=(1, 2))
        fn = jnp.sum(acc_ref[pl.ds(_C, _C)], axis=(1, 2))
        ces = jnp.sum(acc_ref[pl.ds(2 * _C, _C)], axis=(1, 2))
        w = jnp.where(fn > 0, fn, 1.0) / jnp.where(gt > 0, gt, 1.0)
        out_ref[...] = jnp.reshape(jnp.sum(w * ces) / _NPIX, (1, 1))


def _rs(a):
    return jnp.sum(a.reshape(_HB // 8, 8, _W), axis=0)


def kernel(input, target):
    nh = _H // _HB
    grid = (_B * nh,)
    out = pl.pallas_call(
        _loss_kernel,
        grid=grid,
        in_specs=[
            pl.BlockSpec((1, _C, _HB, _W), lambda i: (i // nh, 0, i % nh, 0)),
            pl.BlockSpec((1, _HB, _W), lambda i: (i // nh, i % nh, 0)),
        ],
        out_specs=pl.BlockSpec((1, 1), lambda i: (0, 0)),
        out_shape=jax.ShapeDtypeStruct((1, 1), jnp.float32),
        scratch_shapes=[pltpu.VMEM((3 * _C, 8, _W), jnp.float32)],
        compiler_params=pltpu.CompilerParams(
            dimension_semantics=("arbitrary",),
        ),
    )(input, target)
    return out[0, 0]


# speedup vs baseline: 261.9586x; 1.0215x over previous
import jax
import jax.numpy as jnp
from jax.experimental import pallas as pl
from jax.experimental.pallas import tpu as pltpu

_C = 19
_B, _H, _W = 4, 512, 512
_HB = 64
_NPIX = _B * _H * _W


def _loss_kernel(x_ref, t_ref, out_ref, acc_ref):
    step = pl.program_id(0)

    @pl.when(step == 0)
    def _init():
        acc_ref[...] = jnp.zeros_like(acc_ref)

    t = t_ref[0]
    m = x_ref[0, 0]
    for c in range(1, _C):
        m = jnp.maximum(m, x_ref[0, c])
    first = jnp.full(t.shape, _C, jnp.int32)
    s = jnp.zeros_like(m)
    for c in range(_C - 1, -1, -1):
        v = x_ref[0, c]
        s = s + jnp.exp(v - m)
        first = jnp.where(v == m, c, first)
    lse = jnp.log(s) + m
    mism = (first != t).astype(jnp.float32)
    for c in range(_C):
        maskf = (t == c).astype(jnp.float32)
        acc_ref[c] += _rs(maskf)
        acc_ref[_C + c] += _rs(maskf * mism)
        acc_ref[2 * _C + c] += _rs(maskf * (lse - x_ref[0, c]))

    @pl.when(step == pl.num_programs(0) - 1)
    def _fin():
        gt = jnp.sum(acc_ref[pl.ds(0, _C)], axis=(1, 2))
        fn = jnp.sum(acc_ref[pl.ds(_C, _C)], axis=(1, 2))
        ces = jnp.sum(acc_ref[pl.ds(2 * _C, _C)], axis=(1, 2))
        w = jnp.where(fn > 0, fn, 1.0) / jnp.where(gt > 0, gt, 1.0)
        out_ref[...] = jnp.reshape(jnp.sum(w * ces) / _NPIX, (1, 1))


def _rs(a):
    return jnp.sum(a.reshape(_HB // 8, 8, _W), axis=0)


def kernel(input, target):
    nh = _H // _HB
    grid = (_B * nh,)
    out = pl.pallas_call(
        _loss_kernel,
        grid=grid,
        in_specs=[
            pl.BlockSpec((1, _C, _HB, _W), lambda i: (i // nh, 0, i % nh, 0)),
            pl.BlockSpec((1, _HB, _W), lambda i: (i // nh, i % nh, 0)),
        ],
        out_specs=pl.BlockSpec((1, 1), lambda i: (0, 0)),
        out_shape=jax.ShapeDtypeStruct((1, 1), jnp.float32),
        scratch_shapes=[pltpu.VMEM((3 * _C, 8, _W), jnp.float32)],
        compiler_params=pltpu.CompilerParams(
            dimension_semantics=("arbitrary",),
        ),
    )(input, target)
    return out[0, 0]


# speedup vs baseline: 266.1670x; 1.0161x over previous
import jax
import jax.numpy as jnp
from jax.experimental import pallas as pl
from jax.experimental.pallas import tpu as pltpu

_C = 19
_B, _H, _W = 4, 512, 512
_HB = 64
_NPIX = _B * _H * _W


def _loss_kernel(x_ref, t_ref, out_ref, acc_ref, acci_ref):
    step = pl.program_id(0)

    @pl.when(step == 0)
    def _init():
        acc_ref[...] = jnp.zeros_like(acc_ref)
        acci_ref[...] = jnp.zeros_like(acci_ref)

    t = t_ref[0]
    m = x_ref[0, 0]
    for c in range(1, _C):
        m = jnp.maximum(m, x_ref[0, c])
    first = jnp.full(t.shape, _C, jnp.int32)
    s = jnp.zeros_like(m)
    for c in range(_C - 1, -1, -1):
        v = x_ref[0, c]
        s = s + jnp.exp(v - m)
        first = jnp.where(v == m, c, first)
    lse = jnp.log(s) + m
    t16 = t.astype(jnp.int16)
    mism16 = (first != t).astype(jnp.int16)
    for c in range(_C):
        m16 = (t16 == c).astype(jnp.int16)
        acci_ref[c] += _rs16(m16)
        acci_ref[_C + c] += _rs16(m16 * mism16)
        ce_c = jnp.where(t == c, lse - x_ref[0, c], 0.0)
        acc_ref[c] += _rs(ce_c)

    @pl.when(step == pl.num_programs(0) - 1)
    def _fin():
        gt = jnp.sum(acci_ref[pl.ds(0, _C)].astype(jnp.float32), axis=(1, 2))
        fn = jnp.sum(acci_ref[pl.ds(_C, _C)].astype(jnp.float32), axis=(1, 2))
        ces = jnp.sum(acc_ref[pl.ds(0, _C)], axis=(1, 2))
        w = jnp.where(fn > 0, fn, 1.0) / jnp.where(gt > 0, gt, 1.0)
        out_ref[...] = jnp.reshape(jnp.sum(w * ces) / _NPIX, (1, 1))


def _rs(a):
    return jnp.sum(a.reshape(_HB // 8, 8, _W), axis=0)


def _rs16(a):
    g = a.reshape(_HB // 8, 8, _W)
    r = g[0]
    for i in range(1, _HB // 8):
        r = r + g[i]
    return r


def kernel(input, target):
    nh = _H // _HB
    grid = (_B * nh,)
    out = pl.pallas_call(
        _loss_kernel,
        grid=grid,
        in_specs=[
            pl.BlockSpec((1, _C, _HB, _W), lambda i: (i // nh, 0, i % nh, 0)),
            pl.BlockSpec((1, _HB, _W), lambda i: (i // nh, i % nh, 0)),
        ],
        out_specs=pl.BlockSpec((1, 1), lambda i: (0, 0)),
        out_shape=jax.ShapeDtypeStruct((1, 1), jnp.float32),
        scratch_shapes=[pltpu.VMEM((_C, 8, _W), jnp.float32),
                        pltpu.VMEM((2 * _C, 8, _W), jnp.int16)],
        compiler_params=pltpu.CompilerParams(
            dimension_semantics=("arbitrary",),
        ),
    )(input, target)
    return out[0, 0]
